# baseline (speedup 1.0000x reference)
import jax
import jax.numpy as jnp
from jax import lax
from jax.experimental import pallas as pl
from jax.experimental.pallas import tpu as pltpu
from jax.experimental.pallas import tpu_sc as plsc

N = 10000
E = 320000
D = 128
LAYERS = 3

NC = 2
NS = 16
NW = NC * NS
FPT = D // NW
NPLANE = FPT // 2
FN = FPT * N
CHUNK = 3200
NBLK = CHUNK // 16
NCHUNK = E // CHUNK
NBUF = 4
NVEC = N // 16
UNROLL = 2

_ILV = plsc.PackFormat.INTERLEAVED


def _body(xt_hbm, ed_hbm, out_hbm, y, s, xpk, *rest):
    edb = rest[:NBUF]
    sems = rest[NBUF:]
    cid = lax.axis_index("c")
    sid = lax.axis_index("s")
    wid = cid * NS + sid
    base = wid * FN

    WCH = 2 * CHUNK

    def prime_ring():
        for b in range(NBUF):
            pltpu.async_copy(
                ed_hbm.at[pl.ds(b * WCH, WCH)], edb[b], sems[b])

    pltpu.sync_copy(xt_hbm.at[pl.ds(base, FN)], s)
    prime_ring()

    zeros16 = jnp.zeros((16,), jnp.float32)
    lo16 = jnp.full((16,), 0xFFFF, jnp.int32)

    @pl.loop(0, NVEC)
    def _(i):
        b = i * 16
        for p in range(NPLANE):
            a = s[pl.ds(2 * p * N + b, 16)]
            c = s[pl.ds((2 * p + 1) * N + b, 16)]
            xpk[pl.ds(p * N + b, 16)] = plsc.bitcast(
                plsc.pack(a, c, format=_ILV), jnp.int32)
            y[pl.ds(2 * p * N + b, 16)] = zeros16
            y[pl.ds((2 * p + 1) * N + b, 16)] = zeros16

    def process_chunk(eb):
        @plsc.parallel_loop(0, NBLK // UNROLL)
        def _(j):
            staged = []
            for u in range(UNROLL):
                b = (j * UNROLL + u) * 16
                pk = eb[pl.ds(b, 16)]
                wv = plsc.bitcast(eb[pl.ds(CHUNK + b, 16)], jnp.float32)
                src = pk & lo16
                dst = pk >> 16
                vals = []
                for p in range(NPLANE):
                    xw = plsc.load_gather(xpk.at[pl.ds(p * N, N)], [src])
                    a, c = plsc.unpack(
                        plsc.bitcast(xw, jnp.bfloat16), format=_ILV)
                    vals.append(a * wv)
                    vals.append(c * wv)
                staged.append((dst, vals))
            for dst, vals in staged:
                for f in range(FPT):
                    yf = y.at[pl.ds(f * N, N)]
                    plsc.addupdate_scatter(yf, [dst], vals[f])

    for layer in range(LAYERS):
        @pl.loop(0, NCHUNK - NBUF, step=NBUF)
        def _(c4):
            for b in range(NBUF):
                cc = c4 + b
                e0 = pl.multiple_of(cc * WCH, WCH)
                pltpu.make_async_copy(
                    ed_hbm.at[pl.ds(e0, WCH)], edb[b], sems[b]).wait()
                process_chunk(edb[b])
                e1 = pl.multiple_of((cc + NBUF) * WCH, WCH)
                pltpu.async_copy(
                    ed_hbm.at[pl.ds(e1, WCH)], edb[b], sems[b])

        for b in range(NBUF):
            cc = NCHUNK - NBUF + b
            pltpu.make_async_copy(
                ed_hbm.at[pl.ds(cc * WCH, WCH)], edb[b], sems[b]).wait()
            process_chunk(edb[b])

        if layer < LAYERS - 1:
            prime_ring()

            @pl.loop(0, NVEC)
            def _(i):
                b = i * 16
                for p in range(NPLANE):
                    a = y[pl.ds(2 * p * N + b, 16)]
                    c = y[pl.ds((2 * p + 1) * N + b, 16)]
                    sa = s[pl.ds(2 * p * N + b, 16)]
                    sc = s[pl.ds((2 * p + 1) * N + b, 16)]
                    s[pl.ds(2 * p * N + b, 16)] = sa + a
                    s[pl.ds((2 * p + 1) * N + b, 16)] = sc + c
                    xpk[pl.ds(p * N + b, 16)] = plsc.bitcast(
                        plsc.pack(a, c, format=_ILV), jnp.int32)
                    y[pl.ds(2 * p * N + b, 16)] = zeros16
                    y[pl.ds((2 * p + 1) * N + b, 16)] = zeros16

    quarter = jnp.full((16,), 0.25, jnp.float32)

    @pl.loop(0, NVEC)
    def _(i):
        b = i * 16
        for f in range(FPT):
            ds = pl.ds(f * N + b, 16)
            s[ds] = (s[ds] + y[ds]) * quarter

    pltpu.sync_copy(s, out_hbm.at[pl.ds(base, FN)])


@jax.jit
def _run(xt_flat, edata):
    mesh = plsc.VectorSubcoreMesh(
        core_axis_name="c", subcore_axis_name="s",
        num_cores=NC, num_subcores=NS)
    k = pl.kernel(
        _body,
        out_type=jax.ShapeDtypeStruct((D * N,), jnp.float32),
        mesh=mesh,
        compiler_params=pltpu.CompilerParams(needs_layout_passes=False),
        scratch_types=[
            pltpu.VMEM((FN,), jnp.float32),
            pltpu.VMEM((FN,), jnp.float32),
            pltpu.VMEM((NPLANE * N,), jnp.int32),
        ] + [pltpu.VMEM((2 * CHUNK,), jnp.int32)] * NBUF
          + [pltpu.SemaphoreType.DMA] * NBUF,
    )
    return k(xt_flat, edata)


def kernel(item_embeddings, edge_values, edge_index):
    xt_flat = jnp.transpose(item_embeddings).reshape(D * N)
    src = edge_index[1]
    dst = edge_index[0]
    pk = (dst << 16) | src
    wbits = lax.bitcast_convert_type(edge_values, jnp.int32)
    edata = jnp.concatenate(
        [pk.reshape(NCHUNK, CHUNK), wbits.reshape(NCHUNK, CHUNK)], axis=1
    ).reshape(2 * E)
    out_flat = _run(xt_flat, edata)
    return jnp.transpose(out_flat.reshape(D, N))


# speedup vs baseline: 1.0062x; 1.0062x over previous
import jax
import jax.numpy as jnp
from jax import lax
from jax.experimental import pallas as pl
from jax.experimental.pallas import tpu as pltpu
from jax.experimental.pallas import tpu_sc as plsc

N = 10000
E = 320000
D = 128
LAYERS = 3

NC = 2
NS = 16
NW = NC * NS
FPT = D // NW
NPLANE = FPT // 2
FN = FPT * N
CHUNK = 3200
NBLK = CHUNK // 16
NCHUNK = E // CHUNK
NBUF = 4
NVEC = N // 16
UNROLL = 4

_ILV = plsc.PackFormat.INTERLEAVED


def _body(xt_hbm, ed_hbm, out_hbm, y, s, xpk, *rest):
    edb = rest[:NBUF]
    sems = rest[NBUF:]
    cid = lax.axis_index("c")
    sid = lax.axis_index("s")
    wid = cid * NS + sid
    base = wid * FN

    WCH = 2 * CHUNK

    def prime_ring():
        for b in range(NBUF):
            pltpu.async_copy(
                ed_hbm.at[pl.ds(b * WCH, WCH)], edb[b], sems[b])

    pltpu.sync_copy(xt_hbm.at[pl.ds(base, FN)], s)
    prime_ring()

    zeros16 = jnp.zeros((16,), jnp.float32)
    lo16 = jnp.full((16,), 0xFFFF, jnp.int32)

    @pl.loop(0, NVEC)
    def _(i):
        b = i * 16
        for p in range(NPLANE):
            a = s[pl.ds(2 * p * N + b, 16)]
            c = s[pl.ds((2 * p + 1) * N + b, 16)]
            xpk[pl.ds(p * N + b, 16)] = plsc.bitcast(
                plsc.pack(a, c, format=_ILV), jnp.int32)
            y[pl.ds(2 * p * N + b, 16)] = zeros16
            y[pl.ds((2 * p + 1) * N + b, 16)] = zeros16

    def process_chunk(eb):
        @plsc.parallel_loop(0, NBLK // UNROLL)
        def _(j):
            staged = []
            for u in range(UNROLL):
                b = (j * UNROLL + u) * 16
                pk = eb[pl.ds(b, 16)]
                wv = plsc.bitcast(eb[pl.ds(CHUNK + b, 16)], jnp.float32)
                src = pk & lo16
                dst = pk >> 16
                vals = []
                for p in range(NPLANE):
                    xw = plsc.load_gather(xpk.at[pl.ds(p * N, N)], [src])
                    a, c = plsc.unpack(
                        plsc.bitcast(xw, jnp.bfloat16), format=_ILV)
                    vals.append(a * wv)
                    vals.append(c * wv)
                staged.append((dst, vals))
            for dst, vals in staged:
                for f in range(FPT):
                    yf = y.at[pl.ds(f * N, N)]
                    plsc.addupdate_scatter(yf, [dst], vals[f])

    for layer in range(LAYERS):
        @pl.loop(0, NCHUNK - NBUF, step=NBUF)
        def _(c4):
            for b in range(NBUF):
                cc = c4 + b
                e0 = pl.multiple_of(cc * WCH, WCH)
                pltpu.make_async_copy(
                    ed_hbm.at[pl.ds(e0, WCH)], edb[b], sems[b]).wait()
                process_chunk(edb[b])
                e1 = pl.multiple_of((cc + NBUF) * WCH, WCH)
                pltpu.async_copy(
                    ed_hbm.at[pl.ds(e1, WCH)], edb[b], sems[b])

        for b in range(NBUF):
            cc = NCHUNK - NBUF + b
            pltpu.make_async_copy(
                ed_hbm.at[pl.ds(cc * WCH, WCH)], edb[b], sems[b]).wait()
            process_chunk(edb[b])

        if layer < LAYERS - 1:
            prime_ring()

            @pl.loop(0, NVEC)
            def _(i):
                b = i * 16
                for p in range(NPLANE):
                    a = y[pl.ds(2 * p * N + b, 16)]
                    c = y[pl.ds((2 * p + 1) * N + b, 16)]
                    sa = s[pl.ds(2 * p * N + b, 16)]
                    sc = s[pl.ds((2 * p + 1) * N + b, 16)]
                    s[pl.ds(2 * p * N + b, 16)] = sa + a
                    s[pl.ds((2 * p + 1) * N + b, 16)] = sc + c
                    xpk[pl.ds(p * N + b, 16)] = plsc.bitcast(
                        plsc.pack(a, c, format=_ILV), jnp.int32)
                    y[pl.ds(2 * p * N + b, 16)] = zeros16
                    y[pl.ds((2 * p + 1) * N + b, 16)] = zeros16

    quarter = jnp.full((16,), 0.25, jnp.float32)

    @pl.loop(0, NVEC)
    def _(i):
        b = i * 16
        for f in range(FPT):
            ds = pl.ds(f * N + b, 16)
            s[ds] = (s[ds] + y[ds]) * quarter

    pltpu.sync_copy(s, out_hbm.at[pl.ds(base, FN)])


@jax.jit
def _run(xt_flat, edata):
    mesh = plsc.VectorSubcoreMesh(
        core_axis_name="c", subcore_axis_name="s",
        num_cores=NC, num_subcores=NS)
    k = pl.kernel(
        _body,
        out_type=jax.ShapeDtypeStruct((D * N,), jnp.float32),
        mesh=mesh,
        compiler_params=pltpu.CompilerParams(needs_layout_passes=False),
        scratch_types=[
            pltpu.VMEM((FN,), jnp.float32),
            pltpu.VMEM((FN,), jnp.float32),
            pltpu.VMEM((NPLANE * N,), jnp.int32),
        ] + [pltpu.VMEM((2 * CHUNK,), jnp.int32)] * NBUF
          + [pltpu.SemaphoreType.DMA] * NBUF,
    )
    return k(xt_flat, edata)


def kernel(item_embeddings, edge_values, edge_index):
    xt_flat = jnp.transpose(item_embeddings).reshape(D * N)
    src = edge_index[1]
    dst = edge_index[0]
    pk = (dst << 16) | src
    wbits = lax.bitcast_convert_type(edge_values, jnp.int32)
    edata = jnp.concatenate(
        [pk.reshape(NCHUNK, CHUNK), wbits.reshape(NCHUNK, CHUNK)], axis=1
    ).reshape(2 * E)
    out_flat = _run(xt_flat, edata)
    return jnp.transpose(out_flat.reshape(D, N))
